# baseline (speedup 1.0000x reference)
import functools

import jax
import jax.numpy as jnp
from jax import lax
from jax.experimental import pallas as pl

B = 16384
D = 128
N_NT = 16
R = 16

BLK = 2048


def _fused_body(x_ref, nt_ref, w_ref, idx_ref, o_ref):
    n_iota = lax.broadcasted_iota(jnp.int32, (N_NT, R), 0)
    mask = n_iota == idx_ref[...]
    wv = jnp.sum(jnp.where(mask, w_ref[...], 0.0), axis=1, keepdims=True)
    contrib = jnp.sum(nt_ref[...] * wv.T, axis=1, keepdims=True)
    g = 0.1 + 1.9 * jax.nn.sigmoid(contrib)
    o_ref[...] = x_ref[...] * g


@jax.jit
def kernel(x, nt_levels, w, idx):
    w2 = w.reshape(1, R)
    idx2 = idx.reshape(1, R)
    grid = (B // BLK,)
    return pl.pallas_call(
        _fused_body,
        grid=grid,
        in_specs=[
            pl.BlockSpec((BLK, D), lambda i: (i, 0)),
            pl.BlockSpec((BLK, N_NT), lambda i: (i, 0)),
            pl.BlockSpec((1, R), lambda i: (0, 0)),
            pl.BlockSpec((1, R), lambda i: (0, 0)),
        ],
        out_specs=pl.BlockSpec((BLK, D), lambda i: (i, 0)),
        out_shape=jax.ShapeDtypeStruct((B, D), jnp.float32),
    )(x, nt_levels, w2, idx2)
